# baseline (speedup 1.0000x reference)
import functools

import jax
import jax.numpy as jnp
from jax import lax
from jax.experimental import pallas as pl
from jax.experimental.pallas import tpu as pltpu
from jax.experimental.pallas import tpu_sc as plsc

TOPK = 16
BQ = 256
BK = 4096
NEG = -3.0e38
BIGIDX = 2.0e30
PADVAL = 1.0e4

SC_NC = 2
SC_NS = 16
SC_NW = SC_NC * SC_NS
SC_CHUNK = 512


def _topk_body(nkb, q_ref, k_ref, vals_ref, idx_ref, bv, bi):
    j = pl.program_id(1)

    @pl.when(j == 0)
    def _init():
        bv[...] = jnp.full((BQ, TOPK), NEG, jnp.float32)
        bi[...] = jnp.zeros((BQ, TOPK), jnp.float32)

    q = q_ref[...]
    kt = k_ref[...]
    cross = lax.dot_general(q, kt, (((1,), (1,)), ((), ())),
                            preferred_element_type=jnp.float32)
    qsq = jnp.sum(q * q, axis=1, keepdims=True)
    ksq = jnp.sum(kt * kt, axis=1)[None, :]
    s = -(qsq - 2.0 * cross + ksq)
    ai = lax.broadcasted_iota(jnp.int32, (BQ, BK), 1).astype(jnp.float32)
    jbase = (j * BK).astype(jnp.float32)
    i16 = lax.broadcasted_iota(jnp.int32, (BQ, TOPK), 1).astype(jnp.float32)

    def _cond(carry):
        s_c, mm, b15, bv_c, bi_c = carry
        return jnp.any(mm > b15)

    def _round(carry):
        s_c, mm, b15, bv_c, bi_c = carry
        cand = jnp.where(s_c == mm, ai, BIGIDX)
        am = jnp.min(cand, axis=1, keepdims=True)
        s_c = jnp.where(ai == am, NEG, s_c)
        am = am + jbase
        qual = mm > b15
        pos = jnp.sum(jnp.where(bv_c >= mm, 1.0, 0.0), axis=1, keepdims=True)
        pos = jnp.where(qual, pos, float(TOPK))
        sh_v = jnp.concatenate([mm, bv_c[:, :TOPK - 1]], axis=1)
        sh_i = jnp.concatenate([am, bi_c[:, :TOPK - 1]], axis=1)
        bv_c = jnp.where(i16 < pos, bv_c,
                         jnp.where(i16 == pos, mm, sh_v))
        bi_c = jnp.where(i16 < pos, bi_c,
                         jnp.where(i16 == pos, am, sh_i))
        mm = jnp.max(s_c, axis=1, keepdims=True)
        return s_c, mm, bv_c[:, TOPK - 1:], bv_c, bi_c

    mm0 = jnp.max(s, axis=1, keepdims=True)
    carry = (s, mm0, bv[:, TOPK - 1:], bv[...], bi[...])
    _, _, _, bv_n, bi_n = lax.while_loop(_cond, _round, carry)
    bv[...] = bv_n
    bi[...] = bi_n

    @pl.when(j == nkb - 1)
    def _out():
        vals_ref[...] = bv[...]
        idx_ref[...] = bi[...].astype(jnp.int32)


def _topk_scores(queries, keys_padded, nkb):
    qn = queries.shape[0]
    return pl.pallas_call(
        functools.partial(_topk_body, nkb),
        grid=(qn // BQ, nkb),
        in_specs=[
            pl.BlockSpec((BQ, queries.shape[1]), lambda i, j: (i, 0)),
            pl.BlockSpec((BK, queries.shape[1]), lambda i, j: (j, 0)),
        ],
        out_specs=[
            pl.BlockSpec((BQ, TOPK), lambda i, j: (i, 0)),
            pl.BlockSpec((BQ, TOPK), lambda i, j: (i, 0)),
        ],
        out_shape=[
            jax.ShapeDtypeStruct((qn, TOPK), jnp.float32),
            jax.ShapeDtypeStruct((qn, TOPK), jnp.int32),
        ],
        scratch_shapes=[
            pltpu.VMEM((BQ, TOPK), jnp.float32),
            pltpu.VMEM((BQ, TOPK), jnp.float32),
        ],
    )(queries, keys_padded)


def _sc_merge_body(d, b_per_w, keys_hbm, idx_hbm, q_hbm, out_hbm,
                   idx_v, rows_v, q_v, sem):
    wid = lax.axis_index("s") * SC_NC + lax.axis_index("c")
    nsteps = b_per_w // SC_CHUNK
    qc = SC_CHUNK // TOPK

    for cc in range(nsteps):
        base = wid * b_per_w + cc * SC_CHUNK
        pltpu.sync_copy(idx_hbm.at[pl.ds(base, SC_CHUNK)], idx_v)
        pltpu.async_copy(keys_hbm.at[idx_v], rows_v, sem).wait()
        pltpu.sync_copy(q_hbm.at[pl.ds(wid * (b_per_w // TOPK) + cc * qc, qc)],
                        q_v)

        def _row(i, carry):
            qrow = i // TOPK
            for c4 in range(d // 16):
                g = rows_v[i, pl.ds(c4 * 16, 16)]
                qv16 = q_v[qrow, pl.ds(c4 * 16, 16)]
                rows_v[i, pl.ds(c4 * 16, 16)] = (g + qv16) * 0.5
            return carry

        lax.fori_loop(0, SC_CHUNK, _row, 0)
        pltpu.sync_copy(rows_v, out_hbm.at[pl.ds(base, SC_CHUNK)])


def _sc_merge(keys_wide, idx_flat, queries_wide, d):
    b = idx_flat.shape[0]
    b_per_w = b // SC_NW
    mesh = plsc.VectorSubcoreMesh(core_axis_name="c", subcore_axis_name="s")
    fn = functools.partial(
        pl.kernel,
        mesh=mesh,
        out_type=jax.ShapeDtypeStruct((b, 128), jnp.float32),
        scratch_types=[
            pltpu.VMEM((SC_CHUNK,), jnp.int32),
            pltpu.VMEM((SC_CHUNK, 128), jnp.float32),
            pltpu.VMEM((SC_CHUNK // TOPK, 128), jnp.float32),
            pltpu.SemaphoreType.DMA,
        ],
    )(functools.partial(_sc_merge_body, d, b_per_w))
    return fn(keys_wide, idx_flat, queries_wide)


def kernel(queries, keys, k):
    qn, d = queries.shape
    kn = keys.shape[0]
    nkb = (kn + BK - 1) // BK
    kpad = nkb * BK
    keys_padded = jnp.pad(keys, ((0, kpad - kn), (0, 0)),
                          constant_values=PADVAL)

    vals, idx = _topk_scores(queries, keys_padded, nkb)
    keys_wide = jnp.pad(keys, ((0, 0), (0, 128 - d)))
    queries_wide = jnp.pad(queries, ((0, 0), (0, 128 - d)))
    merged_flat = _sc_merge(keys_wide, idx.reshape(-1), queries_wide, d)
    merged = merged_flat[:, :d].reshape(qn, TOPK, d)

    shift = (jnp.asarray(k) - TOPK).astype(jnp.float32)
    return vals + shift, idx, merged


# speedup vs baseline: 1.1920x; 1.1920x over previous
import functools

import jax
import jax.numpy as jnp
from jax import lax
from jax.experimental import pallas as pl
from jax.experimental.pallas import tpu as pltpu
from jax.experimental.pallas import tpu_sc as plsc

TOPK = 16
BQ = 512
BK = 2048
NEG = -3.0e38
BIGIDX = 2.0e30
PADVAL = 1.0e4

SC_NC = 2
SC_NS = 16
SC_NW = SC_NC * SC_NS
SC_CHUNK = 512


def _topk_body(nkb, q_ref, k_ref, vals_ref, idx_ref, bv, bi):
    j = pl.program_id(1)

    @pl.when(j == 0)
    def _init():
        bv[...] = jnp.full((BQ, TOPK), NEG, jnp.float32)
        bi[...] = jnp.zeros((BQ, TOPK), jnp.float32)

    q = q_ref[...]
    kt = k_ref[...]
    cross = lax.dot_general(q, kt, (((1,), (1,)), ((), ())),
                            preferred_element_type=jnp.float32)
    qsq = jnp.sum(q * q, axis=1, keepdims=True)
    ksq = jnp.sum(kt * kt, axis=1)[None, :]
    s = -(qsq - 2.0 * cross + ksq)
    ai = lax.broadcasted_iota(jnp.int32, (BQ, BK), 1).astype(jnp.float32)
    jbase = (j * BK).astype(jnp.float32)
    i16 = lax.broadcasted_iota(jnp.int32, (BQ, TOPK), 1).astype(jnp.float32)

    def _cond(carry):
        s_c, mm, b15, bv_c, bi_c = carry
        return jnp.any(mm > b15)

    def _round(carry):
        s_c, mm, b15, bv_c, bi_c = carry
        cand = jnp.where(s_c == mm, ai, BIGIDX)
        am = jnp.min(cand, axis=1, keepdims=True)
        s_c = jnp.where(ai == am, NEG, s_c)
        am = am + jbase
        qual = mm > b15
        pos = jnp.sum(jnp.where(bv_c >= mm, 1.0, 0.0), axis=1, keepdims=True)
        pos = jnp.where(qual, pos, float(TOPK))
        sh_v = jnp.concatenate([mm, bv_c[:, :TOPK - 1]], axis=1)
        sh_i = jnp.concatenate([am, bi_c[:, :TOPK - 1]], axis=1)
        bv_c = jnp.where(i16 < pos, bv_c,
                         jnp.where(i16 == pos, mm, sh_v))
        bi_c = jnp.where(i16 < pos, bi_c,
                         jnp.where(i16 == pos, am, sh_i))
        mm = jnp.max(s_c, axis=1, keepdims=True)
        return s_c, mm, bv_c[:, TOPK - 1:], bv_c, bi_c

    mm0 = jnp.max(s, axis=1, keepdims=True)
    carry = (s, mm0, bv[:, TOPK - 1:], bv[...], bi[...])
    _, _, _, bv_n, bi_n = lax.while_loop(_cond, _round, carry)
    bv[...] = bv_n
    bi[...] = bi_n

    @pl.when(j == nkb - 1)
    def _out():
        vals_ref[...] = bv[...]
        idx_ref[...] = bi[...].astype(jnp.int32)


def _topk_scores(queries, keys_padded, nkb):
    qn = queries.shape[0]
    return pl.pallas_call(
        functools.partial(_topk_body, nkb),
        grid=(qn // BQ, nkb),
        in_specs=[
            pl.BlockSpec((BQ, queries.shape[1]), lambda i, j: (i, 0)),
            pl.BlockSpec((BK, queries.shape[1]), lambda i, j: (j, 0)),
        ],
        out_specs=[
            pl.BlockSpec((BQ, TOPK), lambda i, j: (i, 0)),
            pl.BlockSpec((BQ, TOPK), lambda i, j: (i, 0)),
        ],
        out_shape=[
            jax.ShapeDtypeStruct((qn, TOPK), jnp.float32),
            jax.ShapeDtypeStruct((qn, TOPK), jnp.int32),
        ],
        scratch_shapes=[
            pltpu.VMEM((BQ, TOPK), jnp.float32),
            pltpu.VMEM((BQ, TOPK), jnp.float32),
        ],
    )(queries, keys_padded)


def _sc_merge_body(d, b_per_w, keys_hbm, idx_hbm, q_hbm, out_hbm,
                   idx_v, rows_v, q_v, sem):
    wid = lax.axis_index("s") * SC_NC + lax.axis_index("c")
    nsteps = b_per_w // SC_CHUNK
    qc = SC_CHUNK // TOPK

    for cc in range(nsteps):
        base = wid * b_per_w + cc * SC_CHUNK
        pltpu.sync_copy(idx_hbm.at[pl.ds(base, SC_CHUNK)], idx_v)
        pltpu.async_copy(keys_hbm.at[idx_v], rows_v, sem).wait()
        pltpu.sync_copy(q_hbm.at[pl.ds(wid * (b_per_w // TOPK) + cc * qc, qc)],
                        q_v)

        def _row(i, carry):
            qrow = i // TOPK
            for c4 in range(d // 16):
                g = rows_v[i, pl.ds(c4 * 16, 16)]
                qv16 = q_v[qrow, pl.ds(c4 * 16, 16)]
                rows_v[i, pl.ds(c4 * 16, 16)] = (g + qv16) * 0.5
            return carry

        lax.fori_loop(0, SC_CHUNK, _row, 0)
        pltpu.sync_copy(rows_v, out_hbm.at[pl.ds(base, SC_CHUNK)])


def _sc_merge(keys_wide, idx_flat, queries_wide, d):
    b = idx_flat.shape[0]
    b_per_w = b // SC_NW
    mesh = plsc.VectorSubcoreMesh(core_axis_name="c", subcore_axis_name="s")
    fn = functools.partial(
        pl.kernel,
        mesh=mesh,
        out_type=jax.ShapeDtypeStruct((b, 128), jnp.float32),
        scratch_types=[
            pltpu.VMEM((SC_CHUNK,), jnp.int32),
            pltpu.VMEM((SC_CHUNK, 128), jnp.float32),
            pltpu.VMEM((SC_CHUNK // TOPK, 128), jnp.float32),
            pltpu.SemaphoreType.DMA,
        ],
    )(functools.partial(_sc_merge_body, d, b_per_w))
    return fn(keys_wide, idx_flat, queries_wide)


def kernel(queries, keys, k):
    qn, d = queries.shape
    kn = keys.shape[0]
    nkb = (kn + BK - 1) // BK
    kpad = nkb * BK
    keys_padded = jnp.pad(keys, ((0, kpad - kn), (0, 0)),
                          constant_values=PADVAL)

    vals, idx = _topk_scores(queries, keys_padded, nkb)
    keys_wide = jnp.pad(keys, ((0, 0), (0, 128 - d)))
    queries_wide = jnp.pad(queries, ((0, 0), (0, 128 - d)))
    merged_flat = _sc_merge(keys_wide, idx.reshape(-1), queries_wide, d)
    merged = merged_flat[:, :d].reshape(qn, TOPK, d)

    shift = (jnp.asarray(k) - TOPK).astype(jnp.float32)
    return vals + shift, idx, merged
